# Initial kernel scaffold; baseline (speedup 1.0000x reference)
import jax
import jax.numpy as jnp
from jax.experimental import pallas as pl


def kernel(x, edge_index, edge_weight, W1, b1, W2, b2):
    raise NotImplementedError("write your pallas kernel here")


# speedup vs baseline: 8.8743x; 8.8743x over previous
import functools

import jax
import jax.numpy as jnp
from jax import lax
from jax.experimental import pallas as pl
from jax.experimental.pallas import tpu as pltpu
from jax.experimental.pallas import tpu_sc as plsc

_N = 10000
_E = 320000
_NF = 128
_NH = 128
_NCLS = 64

_NC, _NS, _L = 2, 16, 16
_NW = _NC * _NS
_EPT = _E // _NW
_EB = 125
_NB = _EPT // _EB
_RPS = 624
_REM = _N - _RPS * _NS

_MESH = dict(core_axis_name="c", subcore_axis_name="s",
             num_cores=_NC, num_subcores=_NS)


def _stripe_copy(s, src, dst):
    pltpu.sync_copy(src.at[pl.ds(s * _RPS, _RPS)], dst.at[pl.ds(s * _RPS, _RPS)])

    @pl.when(s == _NS - 1)
    def _():
        pltpu.sync_copy(src.at[pl.ds(_RPS * _NS, _REM)],
                        dst.at[pl.ds(_RPS * _NS, _REM)])


_NQ = _N // _L


@functools.partial(
    pl.kernel,
    out_type=jax.ShapeDtypeStruct((_NC, _NQ, _L), jnp.float32),
    mesh=plsc.VectorSubcoreMesh(**_MESH),
    scratch_types=[
        pltpu.VMEM((_NB, _EB), jnp.int32),
        pltpu.VMEM((_EB, _L), jnp.float32),
        pltpu.VMEM_SHARED((_NQ, _L), jnp.float32),
    ],
    compiler_params=pltpu.CompilerParams(use_tc_tiling_on_sc=False),
)
def _deg_call(colq_hbm, ewoh_hbm, z_hbm, out_hbm, colv, bufew, acc):
    c = lax.axis_index("c")
    s = lax.axis_index("s")
    wid = s * _NC + c
    pltpu.sync_copy(colq_hbm.at[wid], colv)

    @pl.when(s == 0)
    def _():
        pltpu.sync_copy(z_hbm, acc)
    plsc.subcore_barrier()

    def batch(j, carry):
        pltpu.sync_copy(ewoh_hbm.at[wid, j], bufew)
        pltpu.sync_copy(bufew, acc.at[colv.at[j]], add=True)
        return carry
    lax.fori_loop(0, _NB, batch, 0)

    plsc.subcore_barrier()

    @pl.when(s == 0)
    def _():
        pltpu.sync_copy(acc, out_hbm.at[c])


def _make_mp(d):
    @functools.partial(
        pl.kernel,
        out_type=jax.ShapeDtypeStruct((_NC, _N, d), jnp.float32),
        mesh=plsc.VectorSubcoreMesh(**_MESH),
        scratch_types=[
            pltpu.VMEM((_NB, _EB), jnp.int32),
            pltpu.VMEM((_NB, _EB), jnp.int32),
            pltpu.VMEM((_EB, _L), jnp.float32),
            pltpu.VMEM((_EB, d), jnp.float32),
            pltpu.VMEM_SHARED((_N, d), jnp.float32),
            pltpu.SemaphoreType.DMA,
        ],
        compiler_params=pltpu.CompilerParams(use_tc_tiling_on_sc=False),
    )
    def mp(g_hbm, row_hbm, col_hbm, ewb_hbm, z_hbm, out_hbm,
           rowv, colv, bufew, buf, acc, sem):
        c = lax.axis_index("c")
        s = lax.axis_index("s")
        wid = s * _NC + c
        pltpu.sync_copy(row_hbm.at[wid], rowv)
        pltpu.sync_copy(col_hbm.at[wid], colv)
        _stripe_copy(s, z_hbm, acc)
        plsc.subcore_barrier()

        nf = d // _L

        def batch(j, carry):
            pltpu.async_copy(g_hbm.at[rowv.at[j]], buf, sem).wait()
            pltpu.sync_copy(ewb_hbm.at[wid, j], bufew)

            def edge(e, carry2):
                w16 = bufew[e]
                for f in range(nf):
                    buf[e, pl.ds(f * _L, _L)] = buf[e, pl.ds(f * _L, _L)] * w16
                return carry2
            lax.fori_loop(0, _EB, edge, 0)

            pltpu.sync_copy(buf, acc.at[colv.at[j]], add=True)
            return carry
        lax.fori_loop(0, _NB, batch, 0)

        plsc.subcore_barrier()
        _stripe_copy(s, acc, out_hbm.at[c])
    return mp


_HW = 64
_mp64 = _make_mp(_HW)


_BR = 1000
_GR = _N // _BR


def _tc1_body(parts_ref, x_ref, w1_ref, dinv_ref, g_ref):
    deg = parts_ref[0] + parts_ref[1] + 1.0
    dinv = jnp.where(deg > 0, lax.rsqrt(deg), 0.0)
    dinv_ref[...] = dinv
    z = jnp.dot(x_ref[...], w1_ref[...], preferred_element_type=jnp.float32)
    g_ref[...] = z * dinv


def _tc1(parts, x, W1):
    return pl.pallas_call(
        _tc1_body,
        grid=(_GR,),
        in_specs=[
            pl.BlockSpec((_NC, _BR, 1), lambda i: (0, i, 0)),
            pl.BlockSpec((_BR, _NF), lambda i: (i, 0)),
            pl.BlockSpec((_NF, _NH), lambda i: (0, 0)),
        ],
        out_specs=[
            pl.BlockSpec((_BR, 1), lambda i: (i, 0)),
            pl.BlockSpec((_BR, _NH), lambda i: (i, 0)),
        ],
        out_shape=[
            jax.ShapeDtypeStruct((_N, 1), jnp.float32),
            jax.ShapeDtypeStruct((_N, _NH), jnp.float32),
        ],
    )(parts, x, W1)


def _tc2_body(s1a_ref, s1b_ref, g_ref, dinv_ref, b1_ref, w2_ref, g2_ref):
    dinv = dinv_ref[...]
    g = g_ref[...]
    b1 = b1_ref[...]
    s1 = jnp.concatenate(
        [s1a_ref[0] + s1a_ref[1], s1b_ref[0] + s1b_ref[1]], axis=1)
    h = jnp.maximum((s1 + g) * dinv + b1, 0.0)
    g2_ref[...] = jnp.dot(
        h, w2_ref[...], preferred_element_type=jnp.float32) * dinv


def _tc2(s1a, s1b, g, dinv, b1r, W2):
    return pl.pallas_call(
        _tc2_body,
        grid=(_GR,),
        in_specs=[
            pl.BlockSpec((_NC, _BR, _HW), lambda i: (0, i, 0)),
            pl.BlockSpec((_NC, _BR, _HW), lambda i: (0, i, 0)),
            pl.BlockSpec((_BR, _NH), lambda i: (i, 0)),
            pl.BlockSpec((_BR, 1), lambda i: (i, 0)),
            pl.BlockSpec((1, _NH), lambda i: (0, 0)),
            pl.BlockSpec((_NH, _NCLS), lambda i: (0, 0)),
        ],
        out_specs=pl.BlockSpec((_BR, _NCLS), lambda i: (i, 0)),
        out_shape=jax.ShapeDtypeStruct((_N, _NCLS), jnp.float32),
    )(s1a, s1b, g, dinv, b1r, W2)


def _tc3_body(s2_ref, g2_ref, dinv_ref, b2_ref, out_ref):
    out_ref[...] = ((s2_ref[0] + s2_ref[1] + g2_ref[...]) * dinv_ref[...]
                    + b2_ref[...])


def _tc3(s2, g2, dinv, b2r):
    return pl.pallas_call(
        _tc3_body,
        grid=(_GR,),
        in_specs=[
            pl.BlockSpec((_NC, _BR, _NCLS), lambda i: (0, i, 0)),
            pl.BlockSpec((_BR, _NCLS), lambda i: (i, 0)),
            pl.BlockSpec((_BR, 1), lambda i: (i, 0)),
            pl.BlockSpec((1, _NCLS), lambda i: (0, 0)),
        ],
        out_specs=pl.BlockSpec((_BR, _NCLS), lambda i: (i, 0)),
        out_shape=jax.ShapeDtypeStruct((_N, _NCLS), jnp.float32),
    )(s2, g2, dinv, b2r)


def kernel(x, edge_index, edge_weight, W1, b1, W2, b2):
    ei = edge_index.astype(jnp.int32)
    row3 = ei[0].reshape(_NW, _NB, _EB)
    col3 = ei[1].reshape(_NW, _NB, _EB)
    ewb = jnp.broadcast_to(
        edge_weight.reshape(_NW, _NB, _EB, 1), (_NW, _NB, _EB, _L))
    ewb = jnp.asarray(ewb)
    colq3 = col3 >> 4
    lane = (col3 & (_L - 1))[..., None]
    ewoh = jnp.where(
        lane == jnp.arange(_L, dtype=jnp.int32), ewb, 0.0)
    zeros_d = jnp.zeros((_NQ, _L), jnp.float32)
    zeros_h = jnp.zeros((_N, _HW), jnp.float32)

    parts = _deg_call(colq3, ewoh, zeros_d)
    dinv, g = _tc1(parts.reshape(_NC, _N, 1), x, W1)
    ga = jnp.asarray(g[:, :_HW])
    gb = jnp.asarray(g[:, _HW:])
    s1a = _mp64(ga, row3, col3, ewb, zeros_h)
    s1b = _mp64(gb, row3, col3, ewb, zeros_h)
    g2 = _tc2(s1a, s1b, g, dinv, b1.reshape(1, _NH), W2)
    s2 = _mp64(g2, row3, col3, ewb, zeros_h)
    out = _tc3(s2, g2, dinv, b2.reshape(1, _NCLS))
    return out


# speedup vs baseline: 13.5999x; 1.5325x over previous
import functools

import jax
import jax.numpy as jnp
from jax import lax
from jax.experimental import pallas as pl
from jax.experimental.pallas import tpu as pltpu
from jax.experimental.pallas import tpu_sc as plsc

_N = 10000
_E = 320000
_NF = 128
_NH = 128
_NCLS = 64

_NC, _NS, _L = 2, 16, 16
_NW = _NC * _NS
_EPT = _E // _NW
_EB = 125
_NB = _EPT // _EB
_RPS = 624
_REM = _N - _RPS * _NS

_MESH = dict(core_axis_name="c", subcore_axis_name="s",
             num_cores=_NC, num_subcores=_NS)


def _stripe_copy(s, src, dst):
    pltpu.sync_copy(src.at[pl.ds(s * _RPS, _RPS)], dst.at[pl.ds(s * _RPS, _RPS)])

    @pl.when(s == _NS - 1)
    def _():
        pltpu.sync_copy(src.at[pl.ds(_RPS * _NS, _REM)],
                        dst.at[pl.ds(_RPS * _NS, _REM)])


_NQ = _N // _L


@functools.partial(
    pl.kernel,
    out_type=jax.ShapeDtypeStruct((_NC, _NQ, _L), jnp.float32),
    mesh=plsc.VectorSubcoreMesh(**_MESH),
    scratch_types=[
        pltpu.VMEM((_NB, _EB), jnp.int32),
        pltpu.VMEM((_EB, _L), jnp.float32),
        pltpu.VMEM_SHARED((_NQ, _L), jnp.float32),
    ],
    compiler_params=pltpu.CompilerParams(use_tc_tiling_on_sc=False),
)
def _deg_call(colq_hbm, ewoh_hbm, z_hbm, out_hbm, colv, bufew, acc):
    c = lax.axis_index("c")
    s = lax.axis_index("s")
    wid = s * _NC + c
    pltpu.sync_copy(colq_hbm.at[wid], colv)

    @pl.when(s == 0)
    def _():
        pltpu.sync_copy(z_hbm, acc)
    plsc.subcore_barrier()

    def batch(j, carry):
        pltpu.sync_copy(ewoh_hbm.at[wid, j], bufew)
        pltpu.sync_copy(bufew, acc.at[colv.at[j]], add=True)
        return carry
    lax.fori_loop(0, _NB, batch, 0)

    plsc.subcore_barrier()

    @pl.when(s == 0)
    def _():
        pltpu.sync_copy(acc, out_hbm.at[c])


def _make_mp(d):
    @functools.partial(
        pl.kernel,
        out_type=jax.ShapeDtypeStruct((_NC, _N, d), jnp.float32),
        mesh=plsc.VectorSubcoreMesh(**_MESH),
        scratch_types=[
            pltpu.VMEM((_NB, _EB), jnp.int32),
            pltpu.VMEM((_NB, _EB), jnp.int32),
            pltpu.VMEM((_EB, _L), jnp.float32),
            pltpu.VMEM((_EB, _L), jnp.float32),
            pltpu.VMEM((_EB, d), jnp.float32),
            pltpu.VMEM((_EB, d), jnp.float32),
            pltpu.VMEM_SHARED((_N, d), jnp.float32),
            pltpu.SemaphoreType.DMA,
            pltpu.SemaphoreType.DMA,
            pltpu.SemaphoreType.DMA,
            pltpu.SemaphoreType.DMA,
        ],
        compiler_params=pltpu.CompilerParams(use_tc_tiling_on_sc=False),
    )
    def mp(g_hbm, row_hbm, col_hbm, ewb_hbm, z_hbm, out_hbm,
           rowv, colv, ew0, ew1, buf0, buf1, acc, sg0, sg1, se0, se1):
        c = lax.axis_index("c")
        s = lax.axis_index("s")
        wid = s * _NC + c
        pltpu.sync_copy(row_hbm.at[wid], rowv)
        pltpu.sync_copy(col_hbm.at[wid], colv)
        _stripe_copy(s, z_hbm, acc)
        plsc.subcore_barrier()

        nf = d // _L
        bufs, ews = (buf0, buf1), (ew0, ew1)
        sgs, ses = (sg0, sg1), (se0, se1)

        def issue(j, b):
            pltpu.async_copy(g_hbm.at[rowv.at[j]], bufs[b], sgs[b])
            pltpu.async_copy(ewb_hbm.at[wid, j], ews[b], ses[b])

        def wait(j, b):
            pltpu.make_async_copy(g_hbm.at[rowv.at[j]], bufs[b], sgs[b]).wait()
            pltpu.make_async_copy(ewb_hbm.at[wid, j], ews[b], ses[b]).wait()

        def scale_scatter(j, b):
            buf, ewb_ = bufs[b], ews[b]

            def edge(e, carry2):
                w16 = ewb_[e]
                for f in range(nf):
                    buf[e, pl.ds(f * _L, _L)] = buf[e, pl.ds(f * _L, _L)] * w16
                return carry2
            lax.fori_loop(0, _EB, edge, 0, unroll=5)
            pltpu.sync_copy(buf, acc.at[colv.at[j]], add=True)

        issue(0, 0)

        def pair(jj, carry):
            j0 = jj * 2
            wait(j0, 0)
            issue(j0 + 1, 1)
            scale_scatter(j0, 0)
            j1 = j0 + 1
            wait(j1, 1)

            @pl.when(j1 + 1 < _NB)
            def _():
                issue(j1 + 1, 0)
            scale_scatter(j1, 1)
            return carry
        lax.fori_loop(0, _NB // 2, pair, 0)

        plsc.subcore_barrier()
        _stripe_copy(s, acc, out_hbm.at[c])
    return mp


_HW = 64
_mp64 = _make_mp(_HW)


_BR = 1000
_GR = _N // _BR


def _tc1_body(parts_ref, x_ref, w1_ref, dinv_ref, g_ref):
    deg = parts_ref[0] + parts_ref[1] + 1.0
    dinv = jnp.where(deg > 0, lax.rsqrt(deg), 0.0)
    dinv_ref[...] = dinv
    z = jnp.dot(x_ref[...], w1_ref[...], preferred_element_type=jnp.float32)
    g_ref[...] = z * dinv


def _tc1(parts, x, W1):
    return pl.pallas_call(
        _tc1_body,
        grid=(_GR,),
        in_specs=[
            pl.BlockSpec((_NC, _BR, 1), lambda i: (0, i, 0)),
            pl.BlockSpec((_BR, _NF), lambda i: (i, 0)),
            pl.BlockSpec((_NF, _NH), lambda i: (0, 0)),
        ],
        out_specs=[
            pl.BlockSpec((_BR, 1), lambda i: (i, 0)),
            pl.BlockSpec((_BR, _NH), lambda i: (i, 0)),
        ],
        out_shape=[
            jax.ShapeDtypeStruct((_N, 1), jnp.float32),
            jax.ShapeDtypeStruct((_N, _NH), jnp.float32),
        ],
    )(parts, x, W1)


def _tc2_body(s1a_ref, s1b_ref, g_ref, dinv_ref, b1_ref, w2_ref, g2_ref):
    dinv = dinv_ref[...]
    g = g_ref[...]
    b1 = b1_ref[...]
    s1 = jnp.concatenate(
        [s1a_ref[0] + s1a_ref[1], s1b_ref[0] + s1b_ref[1]], axis=1)
    h = jnp.maximum((s1 + g) * dinv + b1, 0.0)
    g2_ref[...] = jnp.dot(
        h, w2_ref[...], preferred_element_type=jnp.float32) * dinv


def _tc2(s1a, s1b, g, dinv, b1r, W2):
    return pl.pallas_call(
        _tc2_body,
        grid=(_GR,),
        in_specs=[
            pl.BlockSpec((_NC, _BR, _HW), lambda i: (0, i, 0)),
            pl.BlockSpec((_NC, _BR, _HW), lambda i: (0, i, 0)),
            pl.BlockSpec((_BR, _NH), lambda i: (i, 0)),
            pl.BlockSpec((_BR, 1), lambda i: (i, 0)),
            pl.BlockSpec((1, _NH), lambda i: (0, 0)),
            pl.BlockSpec((_NH, _NCLS), lambda i: (0, 0)),
        ],
        out_specs=pl.BlockSpec((_BR, _NCLS), lambda i: (i, 0)),
        out_shape=jax.ShapeDtypeStruct((_N, _NCLS), jnp.float32),
    )(s1a, s1b, g, dinv, b1r, W2)


def _tc3_body(s2_ref, g2_ref, dinv_ref, b2_ref, out_ref):
    out_ref[...] = ((s2_ref[0] + s2_ref[1] + g2_ref[...]) * dinv_ref[...]
                    + b2_ref[...])


def _tc3(s2, g2, dinv, b2r):
    return pl.pallas_call(
        _tc3_body,
        grid=(_GR,),
        in_specs=[
            pl.BlockSpec((_NC, _BR, _NCLS), lambda i: (0, i, 0)),
            pl.BlockSpec((_BR, _NCLS), lambda i: (i, 0)),
            pl.BlockSpec((_BR, 1), lambda i: (i, 0)),
            pl.BlockSpec((1, _NCLS), lambda i: (0, 0)),
        ],
        out_specs=pl.BlockSpec((_BR, _NCLS), lambda i: (i, 0)),
        out_shape=jax.ShapeDtypeStruct((_N, _NCLS), jnp.float32),
    )(s2, g2, dinv, b2r)


def kernel(x, edge_index, edge_weight, W1, b1, W2, b2):
    ei = edge_index.astype(jnp.int32)
    row3 = ei[0].reshape(_NW, _NB, _EB)
    col3 = ei[1].reshape(_NW, _NB, _EB)
    ewb = jnp.broadcast_to(
        edge_weight.reshape(_NW, _NB, _EB, 1), (_NW, _NB, _EB, _L))
    ewb = jnp.asarray(ewb)
    colq3 = col3 >> 4
    lane = (col3 & (_L - 1))[..., None]
    ewoh = jnp.where(
        lane == jnp.arange(_L, dtype=jnp.int32), ewb, 0.0)
    zeros_d = jnp.zeros((_NQ, _L), jnp.float32)
    zeros_h = jnp.zeros((_N, _HW), jnp.float32)

    parts = _deg_call(colq3, ewoh, zeros_d)
    dinv, g = _tc1(parts.reshape(_NC, _N, 1), x, W1)
    ga = jnp.asarray(g[:, :_HW])
    gb = jnp.asarray(g[:, _HW:])
    s1a = _mp64(ga, row3, col3, ewb, zeros_h)
    s1b = _mp64(gb, row3, col3, ewb, zeros_h)
    g2 = _tc2(s1a, s1b, g, dinv, b1.reshape(1, _NH), W2)
    s2 = _mp64(g2, row3, col3, ewb, zeros_h)
    out = _tc3(s2, g2, dinv, b2.reshape(1, _NCLS))
    return out


# speedup vs baseline: 16.4090x; 1.2066x over previous
import functools

import jax
import jax.numpy as jnp
from jax import lax
from jax.experimental import pallas as pl
from jax.experimental.pallas import tpu as pltpu
from jax.experimental.pallas import tpu_sc as plsc

_N = 10000
_E = 320000
_NF = 128
_NH = 128
_NCLS = 64

_NC, _NS, _L = 2, 16, 16
_NW = _NC * _NS
_EPT = _E // _NW
_EB = 125
_NB = _EPT // _EB
_RPS = 624
_REM = _N - _RPS * _NS

_MESH = dict(core_axis_name="c", subcore_axis_name="s",
             num_cores=_NC, num_subcores=_NS)


def _stripe_copy(s, src, dst):
    pltpu.sync_copy(src.at[pl.ds(s * _RPS, _RPS)], dst.at[pl.ds(s * _RPS, _RPS)])

    @pl.when(s == _NS - 1)
    def _():
        pltpu.sync_copy(src.at[pl.ds(_RPS * _NS, _REM)],
                        dst.at[pl.ds(_RPS * _NS, _REM)])


@functools.partial(
    pl.kernel,
    out_type=jax.ShapeDtypeStruct((_NC, _N, _L), jnp.float32),
    mesh=plsc.VectorSubcoreMesh(**_MESH),
    scratch_types=[
        pltpu.VMEM((_NB, _EB), jnp.int32),
        pltpu.VMEM((_EB, _L), jnp.float32),
        pltpu.VMEM((_EB, _L), jnp.float32),
        pltpu.VMEM_SHARED((_N, _L), jnp.float32),
        pltpu.SemaphoreType.DMA,
        pltpu.SemaphoreType.DMA,
        pltpu.SemaphoreType.DMA,
        pltpu.SemaphoreType.DMA,
    ],
    compiler_params=pltpu.CompilerParams(use_tc_tiling_on_sc=False),
)
def _deg_call(col_hbm, ewb_hbm, z_hbm, out_hbm, colv, ew0, ew1, acc,
              se0, se1, ss0, ss1):
    c = lax.axis_index("c")
    s = lax.axis_index("s")
    wid = s * _NC + c
    pltpu.sync_copy(col_hbm.at[wid], colv)
    _stripe_copy(s, z_hbm, acc)
    plsc.subcore_barrier()

    ews, ses, sss = (ew0, ew1), (se0, se1), (ss0, ss1)

    def issue(j, b):
        pltpu.async_copy(ewb_hbm.at[wid, j], ews[b], ses[b])

    def wait(j, b):
        pltpu.make_async_copy(ewb_hbm.at[wid, j], ews[b], ses[b]).wait()

    def wait_sc(b):
        pltpu.make_async_copy(ews[b], acc.at[pl.ds(0, _EB)], sss[b]).wait()

    issue(0, 0)

    def pair(jj, carry):
        j0 = jj * 2
        wait(j0, 0)

        @pl.when(jj > 0)
        def _():
            wait_sc(1)
        issue(j0 + 1, 1)
        pltpu.async_copy(ew0, acc.at[colv.at[j0]], ss0, add=True)
        j1 = j0 + 1
        wait(j1, 1)
        wait_sc(0)

        @pl.when(j1 + 1 < _NB)
        def _():
            issue(j1 + 1, 0)
        pltpu.async_copy(ew1, acc.at[colv.at[j1]], ss1, add=True)
        return carry
    lax.fori_loop(0, _NB // 2, pair, 0)
    wait_sc(1)

    plsc.subcore_barrier()
    _stripe_copy(s, acc, out_hbm.at[c])


def _make_mp(d):
    @functools.partial(
        pl.kernel,
        out_type=jax.ShapeDtypeStruct((_NC, _N, d), jnp.float32),
        mesh=plsc.VectorSubcoreMesh(**_MESH),
        scratch_types=[
            pltpu.VMEM((_NB, _EB), jnp.int32),
            pltpu.VMEM((_NB, _EB), jnp.int32),
            pltpu.VMEM((_EB, _L), jnp.float32),
            pltpu.VMEM((_EB, _L), jnp.float32),
            pltpu.VMEM((_EB, d), jnp.float32),
            pltpu.VMEM((_EB, d), jnp.float32),
            pltpu.VMEM_SHARED((_N, d), jnp.float32),
            pltpu.SemaphoreType.DMA,
            pltpu.SemaphoreType.DMA,
            pltpu.SemaphoreType.DMA,
            pltpu.SemaphoreType.DMA,
            pltpu.SemaphoreType.DMA,
            pltpu.SemaphoreType.DMA,
        ],
        compiler_params=pltpu.CompilerParams(use_tc_tiling_on_sc=False),
    )
    def mp(g_hbm, row_hbm, col_hbm, ewb_hbm, z_hbm, out_hbm,
           rowv, colv, ew0, ew1, buf0, buf1, acc,
           sg0, sg1, se0, se1, ss0, ss1):
        c = lax.axis_index("c")
        s = lax.axis_index("s")
        wid = s * _NC + c
        pltpu.sync_copy(row_hbm.at[wid], rowv)
        pltpu.sync_copy(col_hbm.at[wid], colv)
        _stripe_copy(s, z_hbm, acc)
        plsc.subcore_barrier()

        nf = d // _L
        bufs, ews = (buf0, buf1), (ew0, ew1)
        sgs, ses, sss = (sg0, sg1), (se0, se1), (ss0, ss1)

        def issue(j, b):
            pltpu.async_copy(g_hbm.at[rowv.at[j]], bufs[b], sgs[b])
            pltpu.async_copy(ewb_hbm.at[wid, j], ews[b], ses[b])

        def wait(j, b):
            pltpu.make_async_copy(g_hbm.at[rowv.at[j]], bufs[b], sgs[b]).wait()
            pltpu.make_async_copy(ewb_hbm.at[wid, j], ews[b], ses[b]).wait()

        def wait_sc(b):
            pltpu.make_async_copy(
                bufs[b], acc.at[pl.ds(0, _EB)], sss[b]).wait()

        def scale_scatter(j, b):
            buf, ewb_ = bufs[b], ews[b]

            def edge(e, carry2):
                w16 = ewb_[e]
                for f in range(nf):
                    buf[e, pl.ds(f * _L, _L)] = buf[e, pl.ds(f * _L, _L)] * w16
                return carry2
            lax.fori_loop(0, _EB, edge, 0, unroll=5)
            pltpu.async_copy(buf, acc.at[colv.at[j]], sss[b], add=True)

        issue(0, 0)

        def pair(jj, carry):
            j0 = jj * 2
            wait(j0, 0)

            @pl.when(jj > 0)
            def _():
                wait_sc(1)
            issue(j0 + 1, 1)
            scale_scatter(j0, 0)
            j1 = j0 + 1
            wait(j1, 1)
            wait_sc(0)

            @pl.when(j1 + 1 < _NB)
            def _():
                issue(j1 + 1, 0)
            scale_scatter(j1, 1)
            return carry
        lax.fori_loop(0, _NB // 2, pair, 0)
        wait_sc(1)

        plsc.subcore_barrier()
        _stripe_copy(s, acc, out_hbm.at[c])
    return mp


_HW = 64
_mp64 = _make_mp(_HW)


_BR = 1000
_GR = _N // _BR


def _tc1_body(parts_ref, x_ref, w1_ref, dinv_ref, g_ref):
    deg = parts_ref[0, :, 0:1] + parts_ref[1, :, 0:1] + 1.0
    dinv = jnp.where(deg > 0, lax.rsqrt(deg), 0.0)
    dinv_ref[...] = dinv
    z = jnp.dot(x_ref[...], w1_ref[...], preferred_element_type=jnp.float32)
    g_ref[...] = z * dinv


def _tc1(parts, x, W1):
    return pl.pallas_call(
        _tc1_body,
        grid=(_GR,),
        in_specs=[
            pl.BlockSpec((_NC, _BR, _L), lambda i: (0, i, 0)),
            pl.BlockSpec((_BR, _NF), lambda i: (i, 0)),
            pl.BlockSpec((_NF, _NH), lambda i: (0, 0)),
        ],
        out_specs=[
            pl.BlockSpec((_BR, 1), lambda i: (i, 0)),
            pl.BlockSpec((_BR, _NH), lambda i: (i, 0)),
        ],
        out_shape=[
            jax.ShapeDtypeStruct((_N, 1), jnp.float32),
            jax.ShapeDtypeStruct((_N, _NH), jnp.float32),
        ],
    )(parts, x, W1)


def _tc2_body(s1a_ref, s1b_ref, g_ref, dinv_ref, b1_ref, w2_ref, g2_ref):
    dinv = dinv_ref[...]
    g = g_ref[...]
    b1 = b1_ref[...]
    s1 = jnp.concatenate(
        [s1a_ref[0] + s1a_ref[1], s1b_ref[0] + s1b_ref[1]], axis=1)
    h = jnp.maximum((s1 + g) * dinv + b1, 0.0)
    g2_ref[...] = jnp.dot(
        h, w2_ref[...], preferred_element_type=jnp.float32) * dinv


def _tc2(s1a, s1b, g, dinv, b1r, W2):
    return pl.pallas_call(
        _tc2_body,
        grid=(_GR,),
        in_specs=[
            pl.BlockSpec((_NC, _BR, _HW), lambda i: (0, i, 0)),
            pl.BlockSpec((_NC, _BR, _HW), lambda i: (0, i, 0)),
            pl.BlockSpec((_BR, _NH), lambda i: (i, 0)),
            pl.BlockSpec((_BR, 1), lambda i: (i, 0)),
            pl.BlockSpec((1, _NH), lambda i: (0, 0)),
            pl.BlockSpec((_NH, _NCLS), lambda i: (0, 0)),
        ],
        out_specs=pl.BlockSpec((_BR, _NCLS), lambda i: (i, 0)),
        out_shape=jax.ShapeDtypeStruct((_N, _NCLS), jnp.float32),
    )(s1a, s1b, g, dinv, b1r, W2)


def _tc3_body(s2_ref, g2_ref, dinv_ref, b2_ref, out_ref):
    out_ref[...] = ((s2_ref[0] + s2_ref[1] + g2_ref[...]) * dinv_ref[...]
                    + b2_ref[...])


def _tc3(s2, g2, dinv, b2r):
    return pl.pallas_call(
        _tc3_body,
        grid=(_GR,),
        in_specs=[
            pl.BlockSpec((_NC, _BR, _NCLS), lambda i: (0, i, 0)),
            pl.BlockSpec((_BR, _NCLS), lambda i: (i, 0)),
            pl.BlockSpec((_BR, 1), lambda i: (i, 0)),
            pl.BlockSpec((1, _NCLS), lambda i: (0, 0)),
        ],
        out_specs=pl.BlockSpec((_BR, _NCLS), lambda i: (i, 0)),
        out_shape=jax.ShapeDtypeStruct((_N, _NCLS), jnp.float32),
    )(s2, g2, dinv, b2r)


def kernel(x, edge_index, edge_weight, W1, b1, W2, b2):
    ei = edge_index.astype(jnp.int32)
    row3 = ei[0].reshape(_NW, _NB, _EB)
    col3 = ei[1].reshape(_NW, _NB, _EB)
    ewb = jnp.broadcast_to(
        edge_weight.reshape(_NW, _NB, _EB, 1), (_NW, _NB, _EB, _L))
    ewb = jnp.asarray(ewb)
    zeros_d = jnp.zeros((_N, _L), jnp.float32)
    zeros_h = jnp.zeros((_N, _HW), jnp.float32)

    parts = _deg_call(col3, ewb, zeros_d)
    dinv, g = _tc1(parts, x, W1)
    ga = jnp.asarray(g[:, :_HW])
    gb = jnp.asarray(g[:, _HW:])
    s1a = _mp64(ga, row3, col3, ewb, zeros_h)
    s1b = _mp64(gb, row3, col3, ewb, zeros_h)
    g2 = _tc2(s1a, s1b, g, dinv, b1.reshape(1, _NH), W2)
    s2 = _mp64(g2, row3, col3, ewb, zeros_h)
    out = _tc3(s2, g2, dinv, b2.reshape(1, _NCLS))
    return out
